# baseline (speedup 1.0000x reference)
import functools

import jax
import jax.numpy as jnp
import numpy as np
from jax import lax
from jax.experimental import pallas as pl
from jax.experimental.pallas import tpu as pltpu
from jax.experimental.pallas import tpu_sc as plsc

NN = 10000
EE = 320000
GG = 16
R = 80
NB = 128
NPASS = 4
NPAD = NB * R
CB = 32
NTILES = 16


def _stage0_body(seq_ref, ns_ref, se_ref, embw_ref, pe_ref, pn_ref, ps_ref,
                 b_ref, out_ref):
    seqb = seq_ref[...]
    ioy = lax.broadcasted_iota(jnp.int32, (1, 21), 1)
    oh = (seqb == ioy).astype(jnp.float32)
    t = jnp.dot(embw_ref[...], pe_ref[...], preferred_element_type=jnp.float32)
    acc = jnp.dot(oh, t, preferred_element_type=jnp.float32)
    acc += jnp.dot(ns_ref[...], pn_ref[...], preferred_element_type=jnp.float32)
    acc += jnp.dot(se_ref[...], ps_ref[...], preferred_element_type=jnp.float32)
    out_ref[...] = acc + b_ref[...]


def _stage0(seq_p, ns_p, se_p, embed_w, pn_w_emb, pn_w_node, pn_w_seq, pn_b):
    grid = NPAD // 256
    return pl.pallas_call(
        _stage0_body,
        grid=(grid,),
        in_specs=[
            pl.BlockSpec((256, 1), lambda i: (i, 0)),
            pl.BlockSpec((256, 6), lambda i: (i, 0)),
            pl.BlockSpec((256, 1280), lambda i: (i, 0)),
            pl.BlockSpec((21, 20), lambda i: (0, 0)),
            pl.BlockSpec((20, 128), lambda i: (0, 0)),
            pl.BlockSpec((6, 128), lambda i: (0, 0)),
            pl.BlockSpec((1280, 128), lambda i: (0, 0)),
            pl.BlockSpec((1, 128), lambda i: (0, 0)),
        ],
        out_specs=pl.BlockSpec((256, 128), lambda i: (i, 0)),
        out_shape=jax.ShapeDtypeStruct((NPAD, 128), jnp.float32),
    )(seq_p, ns_p, se_p, embed_w, pn_w_emb, pn_w_node, pn_w_seq, pn_b)


def _ea_body(es_ref, w_ref, b_ref, out_ref):
    out_ref[...] = (jnp.dot(es_ref[...], w_ref[...],
                            preferred_element_type=jnp.float32) + b_ref[...])


def _stage_ea(edge_s, pe_w, pe_b):
    grid = EE // 512
    return pl.pallas_call(
        _ea_body,
        grid=(grid,),
        in_specs=[
            pl.BlockSpec((512, 39), lambda i: (i, 0)),
            pl.BlockSpec((39, 128), lambda i: (0, 0)),
            pl.BlockSpec((1, 128), lambda i: (0, 0)),
        ],
        out_specs=pl.BlockSpec((512, 128), lambda i: (i, 0)),
        out_shape=jax.ShapeDtypeStruct((EE, 128), jnp.float32),
    )(edge_s, pe_w, pe_b)


def _stageA_body(fo, x_ref, w4_ref, b4_ref, wet_ref, qcat_ref, kv_ref, sk_ref):
    qkvs = jnp.dot(x_ref[...], w4_ref[...],
                   preferred_element_type=jnp.float32) + b4_ref[...]
    q = qkvs[:, :fo]
    qe = jnp.dot(q, wet_ref[...], preferred_element_type=jnp.float32)
    qcat_ref[:, :fo] = q
    qcat_ref[:, fo:] = qe
    kv_ref[...] = qkvs[:, fo:3 * fo]
    sk_ref[...] = qkvs[:, 3 * fo:]


def _stageA(x, w4, b4, wet, fi, fo):
    grid = NPAD // 256
    return pl.pallas_call(
        functools.partial(_stageA_body, fo),
        grid=(grid,),
        in_specs=[
            pl.BlockSpec((256, fi), lambda i: (i, 0)),
            pl.BlockSpec((fi, 4 * fo), lambda i: (0, 0)),
            pl.BlockSpec((1, 4 * fo), lambda i: (0, 0)),
            pl.BlockSpec((fo, 128), lambda i: (0, 0)),
        ],
        out_specs=[
            pl.BlockSpec((256, fo + 128), lambda i: (i, 0)),
            pl.BlockSpec((256, 2 * fo), lambda i: (i, 0)),
            pl.BlockSpec((256, fo), lambda i: (i, 0)),
        ],
        out_shape=[
            jax.ShapeDtypeStruct((NPAD, fo + 128), jnp.float32),
            jax.ShapeDtypeStruct((NPAD, 2 * fo), jnp.float32),
            jax.ShapeDtypeStruct((NPAD, fo), jnp.float32),
        ],
    )(x, w4, b4, wet)


def _stageB_body(av_ref, ae_ref, dn_ref, we_ref, sk_ref, out_ref):
    fix = jnp.dot(ae_ref[...], we_ref[...], preferred_element_type=jnp.float32)
    y = (av_ref[...] + fix) / (dn_ref[...] + 1e-16) + sk_ref[...]
    out_ref[...] = jnp.where(y >= 0, y, 0.01 * y)


def _stageB(accV, accE, den, we, skip, fo):
    grid = NPAD // 256
    return pl.pallas_call(
        _stageB_body,
        grid=(grid,),
        in_specs=[
            pl.BlockSpec((256, fo), lambda i: (i, 0)),
            pl.BlockSpec((256, 128), lambda i: (i, 0)),
            pl.BlockSpec((256, 1), lambda i: (i, 0)),
            pl.BlockSpec((128, fo), lambda i: (0, 0)),
            pl.BlockSpec((256, fo), lambda i: (i, 0)),
        ],
        out_specs=pl.BlockSpec((256, fo), lambda i: (i, 0)),
        out_shape=jax.ShapeDtypeStruct((NPAD, fo), jnp.float32),
    )(accV, accE, den, we, skip)


def _pool_body(x_ref, b_ref, out_ref, cnt_ref):
    pi = pl.program_id(0)

    @pl.when(pi == 0)
    def _():
        out_ref[...] = jnp.zeros_like(out_ref)
        cnt_ref[...] = jnp.zeros_like(cnt_ref)

    bb = b_ref[...]
    ioy = lax.broadcasted_iota(jnp.int32, (1, GG), 1)
    oh = (bb == ioy).astype(jnp.float32)
    out_ref[...] += jnp.dot(oh.T, x_ref[...], preferred_element_type=jnp.float32)
    cnt_ref[...] += jnp.sum(oh, axis=0)[:, None]

    @pl.when(pi == pl.num_programs(0) - 1)
    def _():
        out_ref[...] = out_ref[...] / jnp.maximum(cnt_ref[...], 1.0)


def _pool(x3, batch_p):
    grid = NPAD // 256
    return pl.pallas_call(
        _pool_body,
        grid=(grid,),
        in_specs=[
            pl.BlockSpec((256, 256), lambda i: (i, 0)),
            pl.BlockSpec((256, 1), lambda i: (i, 0)),
        ],
        out_specs=pl.BlockSpec((GG, 256), lambda i: (0, 0)),
        out_shape=jax.ShapeDtypeStruct((GG, 256), jnp.float32),
        scratch_shapes=[pltpu.VMEM((GG, 1), jnp.float32)],
    )(x3, batch_p)


def _sc_edge(qcat, kv, ea, dst_s, src_s, perm_s, bnd2, fo):
    QW = fo + 128
    KW = 2 * fo
    W = ((fo + 128 + 16 + 127) // 128) * 128
    inv = float(1.0 / np.sqrt(fo))
    scmesh = plsc.VectorSubcoreMesh(core_axis_name="c", subcore_axis_name="s")

    @functools.partial(
        pl.kernel,
        out_type=jax.ShapeDtypeStruct((NPAD, W), jnp.float32),
        mesh=scmesh,
        scratch_types=[
            pltpu.VMEM((CB,), jnp.int32),
            pltpu.VMEM((CB,), jnp.int32),
            pltpu.VMEM((CB,), jnp.int32),
            pltpu.VMEM((CB,), jnp.int32),
            pltpu.VMEM((CB,), jnp.int32),
            pltpu.VMEM((CB,), jnp.int32),
            pltpu.VMEM((16,), jnp.int32),
            pltpu.VMEM((CB, QW), jnp.float32),
            pltpu.VMEM((CB, KW), jnp.float32),
            pltpu.VMEM((CB, 128), jnp.float32),
            pltpu.VMEM((CB, QW), jnp.float32),
            pltpu.VMEM((CB, KW), jnp.float32),
            pltpu.VMEM((CB, 128), jnp.float32),
            pltpu.VMEM((R, W), jnp.float32),
            pltpu.SemaphoreType.DMA,
            pltpu.SemaphoreType.DMA,
            pltpu.SemaphoreType.DMA,
            pltpu.SemaphoreType.DMA,
        ],
    )
    def k(qcat_h, kv_h, ea_h, dst_h, src_h, prm_h, bnd_h, out_h,
          i0d, i0s, i0p, i1d, i1s, i1p, bbuf,
          gq0, gkv0, gea0, gq1, gkv1, gea1, acc,
          semi0, semi1, semg0, semg1):
        c = lax.axis_index("c")
        s = lax.axis_index("s")
        wid = c * NTILES + s
        zv = jnp.zeros((16,), jnp.float32)
        lane = lax.broadcasted_iota(jnp.int32, (16,), 0)
        den_mask = jnp.where(lane == jnp.zeros((16,), jnp.int32),
                             jnp.full((16,), 1.0, jnp.float32),
                             jnp.zeros((16,), jnp.float32))
        invv = jnp.full((16,), inv, jnp.float32)
        m60 = jnp.full((16,), -60.0, jnp.float32)
        p60 = jnp.full((16,), 60.0, jnp.float32)
        zvf = jnp.zeros((16,), jnp.float32)
        ziv = jnp.zeros((16,), jnp.int32)
        rm1 = jnp.full((16,), R - 1, jnp.int32)
        rv = jnp.full((16,), R, jnp.int32)
        m15 = jnp.full((16,), 15, jnp.int32)
        perms = [lane ^ jnp.full((16,), kk2, jnp.int32) for kk2 in (1, 2, 4, 8)]
        ibufs = ((i0d, i0s, i0p), (i1d, i1s, i1p))
        gbufs = ((gq0, gkv0, gea0), (gq1, gkv1, gea1))
        semi = (semi0, semi1)
        semg = (semg0, semg1)

        def issue_idx(ci, b):
            base = ci * CB
            pltpu.make_async_copy(dst_h.at[pl.ds(base, CB)],
                                  ibufs[b][0], semi[b]).start()
            pltpu.make_async_copy(src_h.at[pl.ds(base, CB)],
                                  ibufs[b][1], semi[b]).start()
            pltpu.make_async_copy(prm_h.at[pl.ds(base, CB)],
                                  ibufs[b][2], semi[b]).start()

        def wait_idx(b):
            for j3 in range(3):
                pltpu.make_async_copy(dst_h.at[pl.ds(0, CB)],
                                      ibufs[b][j3], semi[b]).wait()

        def issue_gathers(b):
            pltpu.make_async_copy(qcat_h.at[ibufs[b][0]], gbufs[b][0],
                                  semg[b]).start()
            pltpu.make_async_copy(kv_h.at[ibufs[b][1]], gbufs[b][1],
                                  semg[b]).start()
            pltpu.make_async_copy(ea_h.at[ibufs[b][2]], gbufs[b][2],
                                  semg[b]).start()

        def wait_gathers(b):
            pltpu.make_async_copy(qcat_h.at[ibufs[b][0]], gbufs[b][0],
                                  semg[b]).wait()
            pltpu.make_async_copy(kv_h.at[ibufs[b][1]], gbufs[b][1],
                                  semg[b]).wait()
            pltpu.make_async_copy(ea_h.at[ibufs[b][2]], gbufs[b][2],
                                  semg[b]).wait()

        for p in range(NPASS):
            b_ = 32 * p + wid
            lo_node = b_ * R

            def zrow(r, _):
                for j in range(W // 16):
                    acc[r, pl.ds(16 * j, 16)] = zv
                return 0
            lax.fori_loop(0, R, zrow, 0)

            pltpu.sync_copy(bnd_h.at[b_], bbuf)
            bv = bbuf[...]
            blo, bhi = bv[0], bv[1]
            lo_c = blo // CB
            hi_c = (bhi + CB - 1) // CB
            kmax = jnp.maximum(hi_c - lo_c, 0)
            lov = jnp.full((16,), lo_node, jnp.int32)
            hiv = lov + rv

            @pl.when(kmax > 0)
            def _():
                issue_idx(lo_c, 0)
                wait_idx(0)
                issue_gathers(0)

            @pl.when(kmax > 1)
            def _():
                issue_idx(lo_c + 1, 1)

            def compute(b):
                dbuf = ibufs[b][0]
                gq, gkv, gea = gbufs[b]

                def group(g, _):
                    ebase = 16 * g

                    def edot(l2, av):
                        for u in range(2):
                            l = 2 * l2 + u
                            e = ebase + l
                            a16 = gq[e, pl.ds(0, 16)] * gkv[e, pl.ds(0, 16)]
                            for j in range(1, fo // 16):
                                a16 += (gq[e, pl.ds(16 * j, 16)]
                                        * gkv[e, pl.ds(16 * j, 16)])
                            for j in range(8):
                                a16 += (gq[e, pl.ds(fo + 16 * j, 16)]
                                        * gea[e, pl.ds(16 * j, 16)])
                            a16 = a16 * invv
                            for pv in perms:
                                a16 = a16 + a16.at[pv].get(
                                    mode="promise_in_bounds")
                            lv = jnp.full((16,), l, jnp.int32)
                            av = jnp.where(lane == lv, a16, av)
                        return av
                    av = lax.fori_loop(0, 8, edot, zvf)
                    av = jnp.minimum(jnp.maximum(av, m60), p60)
                    ex = jnp.exp(av)
                    dv = dbuf[pl.ds(ebase, 16)]
                    ok = (dv >= lov) & (dv < hiv)
                    exm = jnp.where(ok, ex, zvf)
                    li = jnp.minimum(jnp.maximum(dv - lov, ziv), rm1)

                    def erow(l2, _):
                        for u in range(2):
                            l = 2 * l2 + u
                            e = ebase + l
                            lv = jnp.full((16,), l, jnp.int32)
                            bex = exm.at[lv].get(mode="promise_in_bounds")
                            perm_l = (lane + lv) & m15
                            r = li.at[perm_l].get(mode="promise_in_bounds")[0]
                            for j in range(fo // 16):
                                acc[r, pl.ds(16 * j, 16)] += (
                                    gkv[e, pl.ds(fo + 16 * j, 16)] * bex)
                            for j in range(8):
                                acc[r, pl.ds(fo + 16 * j, 16)] += (
                                    gea[e, pl.ds(16 * j, 16)] * bex)
                            acc[r, pl.ds(fo + 128, 16)] += bex * den_mask
                        return 0
                    lax.fori_loop(0, 8, erow, 0)
                    return 0
                lax.fori_loop(0, CB // 16, group, 0)

            def pair(t2, _):
                for b in range(2):
                    t = 2 * t2 + b

                    @pl.when(t < kmax)
                    def _():
                        wait_gathers(b)

                        @pl.when(t + 1 < kmax)
                        def _():
                            wait_idx(1 - b)
                            issue_gathers(1 - b)
                        compute(b)

                        @pl.when(t + 2 < kmax)
                        def _():
                            issue_idx(lo_c + t + 2, b)
                return 0
            lax.fori_loop(0, (kmax + 1) // 2, pair, 0)
            pltpu.sync_copy(acc, out_h.at[pl.ds(lo_node, R)])

    return k(qcat, kv, ea, dst_s, src_s, perm_s, bnd2)


def _pad_nodes(a):
    out = jnp.zeros((NPAD,) + a.shape[1:], a.dtype)
    return out.at[:NN].set(a)


def kernel(seq, node_s, seq_emb, edge_index, edge_s, batch, embed_w,
           pn_w, pn_b, pe_w, pe_b, l1, l2, l3):
    src = edge_index[0].astype(jnp.int32)
    dst = edge_index[1].astype(jnp.int32)

    perm = jnp.argsort(dst).astype(jnp.int32)
    dst_s = dst[perm]
    src_s = src[perm]
    bounds = jnp.searchsorted(dst_s, jnp.arange(NB + 1, dtype=jnp.int32) * R
                              ).astype(jnp.int32)
    bnd2 = jnp.zeros((NB, 16), jnp.int32)
    bnd2 = bnd2.at[:, 0].set(bounds[:NB]).at[:, 1].set(bounds[1:])

    seq_p = _pad_nodes(seq.astype(jnp.int32))[:, None]
    ns_p = _pad_nodes(node_s)
    se_p = _pad_nodes(seq_emb)
    batch_p = (jnp.full((NPAD,), GG, jnp.int32)
               .at[:NN].set(batch.astype(jnp.int32)))[:, None]

    x = _stage0(seq_p, ns_p, se_p, embed_w,
                pn_w[:20], pn_w[20:26], pn_w[26:], pn_b[None, :])
    ea = _stage_ea(edge_s, pe_w, pe_b[None, :])

    fi = 128
    for p in (l1, l2, l3):
        fo = p['wq'].shape[1]
        w4 = jnp.concatenate([p['wq'], p['wk'], p['wv'], p['ws']], axis=1)
        b4 = jnp.concatenate([p['bq'], p['bk'], p['bv'], p['bs']])[None, :]
        qcat, kv, skip = _stageA(x, w4, b4, p['we'].T, fi, fo)
        acc = _sc_edge(qcat, kv, ea, dst_s, src_s, perm, bnd2, fo)
        W = ((fo + 128 + 16 + 127) // 128) * 128
        x = _stageB(acc[:, :fo], acc[:, fo:fo + 128],
                    acc[:, fo + 128:fo + 129], p['we'], skip, fo)
        fi = fo

    pooled = _pool(x, batch_p)
    return pooled


# speedup vs baseline: 1.0221x; 1.0221x over previous
import functools

import jax
import jax.numpy as jnp
import numpy as np
from jax import lax
from jax.experimental import pallas as pl
from jax.experimental.pallas import tpu as pltpu
from jax.experimental.pallas import tpu_sc as plsc

NN = 10000
EE = 320000
GG = 16
R = 80
NB = 128
NPASS = 4
NPAD = NB * R
CB = 32
NTILES = 16


def _stage0_body(seq_ref, ns_ref, se_ref, embw_ref, pe_ref, pn_ref, ps_ref,
                 b_ref, out_ref):
    seqb = seq_ref[...]
    ioy = lax.broadcasted_iota(jnp.int32, (1, 21), 1)
    oh = (seqb == ioy).astype(jnp.float32)
    t = jnp.dot(embw_ref[...], pe_ref[...], preferred_element_type=jnp.float32)
    acc = jnp.dot(oh, t, preferred_element_type=jnp.float32)
    acc += jnp.dot(ns_ref[...], pn_ref[...], preferred_element_type=jnp.float32)
    acc += jnp.dot(se_ref[...], ps_ref[...], preferred_element_type=jnp.float32)
    out_ref[...] = acc + b_ref[...]


def _stage0(seq_p, ns_p, se_p, embed_w, pn_w_emb, pn_w_node, pn_w_seq, pn_b):
    grid = NPAD // 256
    return pl.pallas_call(
        _stage0_body,
        grid=(grid,),
        in_specs=[
            pl.BlockSpec((256, 1), lambda i: (i, 0)),
            pl.BlockSpec((256, 6), lambda i: (i, 0)),
            pl.BlockSpec((256, 1280), lambda i: (i, 0)),
            pl.BlockSpec((21, 20), lambda i: (0, 0)),
            pl.BlockSpec((20, 128), lambda i: (0, 0)),
            pl.BlockSpec((6, 128), lambda i: (0, 0)),
            pl.BlockSpec((1280, 128), lambda i: (0, 0)),
            pl.BlockSpec((1, 128), lambda i: (0, 0)),
        ],
        out_specs=pl.BlockSpec((256, 128), lambda i: (i, 0)),
        out_shape=jax.ShapeDtypeStruct((NPAD, 128), jnp.float32),
    )(seq_p, ns_p, se_p, embed_w, pn_w_emb, pn_w_node, pn_w_seq, pn_b)


def _ea_body(es_ref, w_ref, b_ref, out_ref):
    out_ref[...] = (jnp.dot(es_ref[...], w_ref[...],
                            preferred_element_type=jnp.float32) + b_ref[...])


def _stage_ea(edge_s, pe_w, pe_b):
    grid = EE // 512
    return pl.pallas_call(
        _ea_body,
        grid=(grid,),
        in_specs=[
            pl.BlockSpec((512, 39), lambda i: (i, 0)),
            pl.BlockSpec((39, 128), lambda i: (0, 0)),
            pl.BlockSpec((1, 128), lambda i: (0, 0)),
        ],
        out_specs=pl.BlockSpec((512, 128), lambda i: (i, 0)),
        out_shape=jax.ShapeDtypeStruct((EE, 128), jnp.float32),
    )(edge_s, pe_w, pe_b)


def _stageA_body(fo, x_ref, w4_ref, b4_ref, wet_ref, qcat_ref, kv_ref, sk_ref):
    qkvs = jnp.dot(x_ref[...], w4_ref[...],
                   preferred_element_type=jnp.float32) + b4_ref[...]
    q = qkvs[:, :fo]
    qe = jnp.dot(q, wet_ref[...], preferred_element_type=jnp.float32)
    qcat_ref[:, :fo] = q
    qcat_ref[:, fo:] = qe
    kv_ref[...] = qkvs[:, fo:3 * fo]
    sk_ref[...] = qkvs[:, 3 * fo:]


def _stageA(x, w4, b4, wet, fi, fo):
    grid = NPAD // 256
    return pl.pallas_call(
        functools.partial(_stageA_body, fo),
        grid=(grid,),
        in_specs=[
            pl.BlockSpec((256, fi), lambda i: (i, 0)),
            pl.BlockSpec((fi, 4 * fo), lambda i: (0, 0)),
            pl.BlockSpec((1, 4 * fo), lambda i: (0, 0)),
            pl.BlockSpec((fo, 128), lambda i: (0, 0)),
        ],
        out_specs=[
            pl.BlockSpec((256, fo + 128), lambda i: (i, 0)),
            pl.BlockSpec((256, 2 * fo), lambda i: (i, 0)),
            pl.BlockSpec((256, fo), lambda i: (i, 0)),
        ],
        out_shape=[
            jax.ShapeDtypeStruct((NPAD, fo + 128), jnp.float32),
            jax.ShapeDtypeStruct((NPAD, 2 * fo), jnp.float32),
            jax.ShapeDtypeStruct((NPAD, fo), jnp.float32),
        ],
    )(x, w4, b4, wet)


def _stageB_body(av_ref, ae_ref, dn_ref, we_ref, sk_ref, out_ref):
    fix = jnp.dot(ae_ref[...], we_ref[...], preferred_element_type=jnp.float32)
    y = (av_ref[...] + fix) / (dn_ref[...] + 1e-16) + sk_ref[...]
    out_ref[...] = jnp.where(y >= 0, y, 0.01 * y)


def _stageB(accV, accE, den, we, skip, fo):
    grid = NPAD // 256
    return pl.pallas_call(
        _stageB_body,
        grid=(grid,),
        in_specs=[
            pl.BlockSpec((256, fo), lambda i: (i, 0)),
            pl.BlockSpec((256, 128), lambda i: (i, 0)),
            pl.BlockSpec((256, 1), lambda i: (i, 0)),
            pl.BlockSpec((128, fo), lambda i: (0, 0)),
            pl.BlockSpec((256, fo), lambda i: (i, 0)),
        ],
        out_specs=pl.BlockSpec((256, fo), lambda i: (i, 0)),
        out_shape=jax.ShapeDtypeStruct((NPAD, fo), jnp.float32),
    )(accV, accE, den, we, skip)


def _pool_body(x_ref, b_ref, out_ref, cnt_ref):
    pi = pl.program_id(0)

    @pl.when(pi == 0)
    def _():
        out_ref[...] = jnp.zeros_like(out_ref)
        cnt_ref[...] = jnp.zeros_like(cnt_ref)

    bb = b_ref[...]
    ioy = lax.broadcasted_iota(jnp.int32, (1, GG), 1)
    oh = (bb == ioy).astype(jnp.float32)
    out_ref[...] += jnp.dot(oh.T, x_ref[...], preferred_element_type=jnp.float32)
    cnt_ref[...] += jnp.sum(oh, axis=0)[:, None]

    @pl.when(pi == pl.num_programs(0) - 1)
    def _():
        out_ref[...] = out_ref[...] / jnp.maximum(cnt_ref[...], 1.0)


def _pool(x3, batch_p):
    grid = NPAD // 256
    return pl.pallas_call(
        _pool_body,
        grid=(grid,),
        in_specs=[
            pl.BlockSpec((256, 256), lambda i: (i, 0)),
            pl.BlockSpec((256, 1), lambda i: (i, 0)),
        ],
        out_specs=pl.BlockSpec((GG, 256), lambda i: (0, 0)),
        out_shape=jax.ShapeDtypeStruct((GG, 256), jnp.float32),
        scratch_shapes=[pltpu.VMEM((GG, 1), jnp.float32)],
    )(x3, batch_p)


def _sc_edge(qcat, kv, ea, dst_s, src_s, perm_s, bnd2, fo):
    QW = fo + 128
    KW = 2 * fo
    W = ((fo + 128 + 16 + 127) // 128) * 128
    inv = float(1.0 / np.sqrt(fo))
    scmesh = plsc.VectorSubcoreMesh(core_axis_name="c", subcore_axis_name="s")

    @functools.partial(
        pl.kernel,
        out_type=jax.ShapeDtypeStruct((NPAD, W), jnp.float32),
        mesh=scmesh,
        scratch_types=[
            pltpu.VMEM((CB,), jnp.int32),
            pltpu.VMEM((CB,), jnp.int32),
            pltpu.VMEM((CB,), jnp.int32),
            pltpu.VMEM((CB,), jnp.int32),
            pltpu.VMEM((CB,), jnp.int32),
            pltpu.VMEM((CB,), jnp.int32),
            pltpu.VMEM((16,), jnp.int32),
            pltpu.VMEM((CB, QW), jnp.float32),
            pltpu.VMEM((CB, KW), jnp.float32),
            pltpu.VMEM((CB, 128), jnp.float32),
            pltpu.VMEM((CB, QW), jnp.float32),
            pltpu.VMEM((CB, KW), jnp.float32),
            pltpu.VMEM((CB, 128), jnp.float32),
            pltpu.VMEM((R, W), jnp.float32),
            pltpu.SemaphoreType.DMA,
            pltpu.SemaphoreType.DMA,
            pltpu.SemaphoreType.DMA,
            pltpu.SemaphoreType.DMA,
        ],
    )
    def k(qcat_h, kv_h, ea_h, dst_h, src_h, prm_h, bnd_h, out_h,
          i0d, i0s, i0p, i1d, i1s, i1p, bbuf,
          gq0, gkv0, gea0, gq1, gkv1, gea1, acc,
          semi0, semi1, semg0, semg1):
        c = lax.axis_index("c")
        s = lax.axis_index("s")
        wid = c * NTILES + s
        zv = jnp.zeros((16,), jnp.float32)
        lane = lax.broadcasted_iota(jnp.int32, (16,), 0)
        den_mask = jnp.where(lane == jnp.zeros((16,), jnp.int32),
                             jnp.full((16,), 1.0, jnp.float32),
                             jnp.zeros((16,), jnp.float32))
        invv = jnp.full((16,), inv, jnp.float32)
        m60 = jnp.full((16,), -60.0, jnp.float32)
        p60 = jnp.full((16,), 60.0, jnp.float32)
        zvf = jnp.zeros((16,), jnp.float32)
        ziv = jnp.zeros((16,), jnp.int32)
        rm1 = jnp.full((16,), R - 1, jnp.int32)
        rv = jnp.full((16,), R, jnp.int32)
        m15 = jnp.full((16,), 15, jnp.int32)
        perms = [lane ^ jnp.full((16,), kk2, jnp.int32) for kk2 in (1, 2, 4, 8)]
        ibufs = ((i0d, i0s, i0p), (i1d, i1s, i1p))
        gbufs = ((gq0, gkv0, gea0), (gq1, gkv1, gea1))
        semi = (semi0, semi1)
        semg = (semg0, semg1)

        def issue_idx(ci, b):
            base = ci * CB
            pltpu.make_async_copy(dst_h.at[pl.ds(base, CB)],
                                  ibufs[b][0], semi[b]).start()
            pltpu.make_async_copy(src_h.at[pl.ds(base, CB)],
                                  ibufs[b][1], semi[b]).start()
            pltpu.make_async_copy(prm_h.at[pl.ds(base, CB)],
                                  ibufs[b][2], semi[b]).start()

        def wait_idx(b):
            for j3 in range(3):
                pltpu.make_async_copy(dst_h.at[pl.ds(0, CB)],
                                      ibufs[b][j3], semi[b]).wait()

        def issue_gathers(b):
            pltpu.make_async_copy(qcat_h.at[ibufs[b][0]], gbufs[b][0],
                                  semg[b]).start()
            pltpu.make_async_copy(kv_h.at[ibufs[b][1]], gbufs[b][1],
                                  semg[b]).start()
            pltpu.make_async_copy(ea_h.at[ibufs[b][2]], gbufs[b][2],
                                  semg[b]).start()

        def wait_gathers(b):
            pltpu.make_async_copy(qcat_h.at[ibufs[b][0]], gbufs[b][0],
                                  semg[b]).wait()
            pltpu.make_async_copy(kv_h.at[ibufs[b][1]], gbufs[b][1],
                                  semg[b]).wait()
            pltpu.make_async_copy(ea_h.at[ibufs[b][2]], gbufs[b][2],
                                  semg[b]).wait()

        for p in range(NPASS):
            b_ = 32 * p + wid
            lo_node = b_ * R

            def zrow(r, _):
                for j in range(W // 16):
                    acc[r, pl.ds(16 * j, 16)] = zv
                return 0
            lax.fori_loop(0, R, zrow, 0)

            pltpu.sync_copy(bnd_h.at[b_], bbuf)
            bv = bbuf[...]
            blo, bhi = bv[0], bv[1]
            lo_c = blo // CB
            hi_c = (bhi + CB - 1) // CB
            kmax = jnp.maximum(hi_c - lo_c, 0)
            lov = jnp.full((16,), lo_node, jnp.int32)
            hiv = lov + rv

            @pl.when(kmax > 0)
            def _():
                issue_idx(lo_c, 0)
                wait_idx(0)
                issue_gathers(0)

            @pl.when(kmax > 1)
            def _():
                issue_idx(lo_c + 1, 1)

            def compute(b):
                dbuf = ibufs[b][0]
                gq, gkv, gea = gbufs[b]

                def group(g, _):
                    ebase = 16 * g

                    def edot(l, av):
                        e = ebase + l
                        a16 = gq[e, pl.ds(0, 16)] * gkv[e, pl.ds(0, 16)]
                        for j in range(1, fo // 16):
                            a16 += (gq[e, pl.ds(16 * j, 16)]
                                    * gkv[e, pl.ds(16 * j, 16)])
                        for j in range(8):
                            a16 += (gq[e, pl.ds(fo + 16 * j, 16)]
                                    * gea[e, pl.ds(16 * j, 16)])
                        a16 = a16 * invv
                        for pv in perms:
                            a16 = a16 + a16.at[pv].get(mode="promise_in_bounds")
                        lv = jnp.full((16,), l, jnp.int32)
                        return jnp.where(lane == lv, a16, av)
                    av = lax.fori_loop(0, 16, edot, zvf)
                    av = jnp.minimum(jnp.maximum(av, m60), p60)
                    ex = jnp.exp(av)
                    dv = dbuf[pl.ds(ebase, 16)]
                    ok = (dv >= lov) & (dv < hiv)
                    exm = jnp.where(ok, ex, zvf)
                    li = jnp.minimum(jnp.maximum(dv - lov, ziv), rm1)

                    def erow(l, _):
                        e = ebase + l
                        lv = jnp.full((16,), l, jnp.int32)
                        bex = exm.at[lv].get(mode="promise_in_bounds")
                        perm_l = (lane + lv) & m15
                        r = li.at[perm_l].get(mode="promise_in_bounds")[0]
                        for j in range(fo // 16):
                            acc[r, pl.ds(16 * j, 16)] += (
                                gkv[e, pl.ds(fo + 16 * j, 16)] * bex)
                        for j in range(8):
                            acc[r, pl.ds(fo + 16 * j, 16)] += (
                                gea[e, pl.ds(16 * j, 16)] * bex)
                        acc[r, pl.ds(fo + 128, 16)] += bex * den_mask
                        return 0
                    lax.fori_loop(0, 16, erow, 0)
                    return 0
                lax.fori_loop(0, CB // 16, group, 0)

            def pair(t2, _):
                for b in range(2):
                    t = 2 * t2 + b

                    @pl.when(t < kmax)
                    def _():
                        wait_gathers(b)

                        @pl.when(t + 1 < kmax)
                        def _():
                            wait_idx(1 - b)
                            issue_gathers(1 - b)
                        compute(b)

                        @pl.when(t + 2 < kmax)
                        def _():
                            issue_idx(lo_c + t + 2, b)
                return 0
            lax.fori_loop(0, (kmax + 1) // 2, pair, 0)
            pltpu.sync_copy(acc, out_h.at[pl.ds(lo_node, R)])

    return k(qcat, kv, ea, dst_s, src_s, perm_s, bnd2)


def _pad_nodes(a):
    out = jnp.zeros((NPAD,) + a.shape[1:], a.dtype)
    return out.at[:NN].set(a)


def kernel(seq, node_s, seq_emb, edge_index, edge_s, batch, embed_w,
           pn_w, pn_b, pe_w, pe_b, l1, l2, l3):
    src = edge_index[0].astype(jnp.int32)
    dst = edge_index[1].astype(jnp.int32)

    perm = jnp.argsort(dst).astype(jnp.int32)
    dst_s = dst[perm]
    src_s = src[perm]
    bounds = jnp.searchsorted(dst_s, jnp.arange(NB + 1, dtype=jnp.int32) * R
                              ).astype(jnp.int32)
    bnd2 = jnp.zeros((NB, 16), jnp.int32)
    bnd2 = bnd2.at[:, 0].set(bounds[:NB]).at[:, 1].set(bounds[1:])

    seq_p = _pad_nodes(seq.astype(jnp.int32))[:, None]
    ns_p = _pad_nodes(node_s)
    se_p = _pad_nodes(seq_emb)
    batch_p = (jnp.full((NPAD,), GG, jnp.int32)
               .at[:NN].set(batch.astype(jnp.int32)))[:, None]

    x = _stage0(seq_p, ns_p, se_p, embed_w,
                pn_w[:20], pn_w[20:26], pn_w[26:], pn_b[None, :])
    ea = _stage_ea(edge_s, pe_w, pe_b[None, :])

    fi = 128
    for p in (l1, l2, l3):
        fo = p['wq'].shape[1]
        w4 = jnp.concatenate([p['wq'], p['wk'], p['wv'], p['ws']], axis=1)
        b4 = jnp.concatenate([p['bq'], p['bk'], p['bv'], p['bs']])[None, :]
        qcat, kv, skip = _stageA(x, w4, b4, p['we'].T, fi, fo)
        acc = _sc_edge(qcat, kv, ea, dst_s, src_s, perm, bnd2, fo)
        W = ((fo + 128 + 16 + 127) // 128) * 128
        x = _stageB(acc[:, :fo], acc[:, fo:fo + 128],
                    acc[:, fo + 128:fo + 129], p['we'], skip, fo)
        fi = fo

    pooled = _pool(x, batch_p)
    return pooled
